# baseline (speedup 1.0000x reference)
import functools

import jax
import jax.numpy as jnp
from jax import lax
from jax.experimental import pallas as pl
from jax.experimental.pallas import tpu as pltpu
from jax.experimental.pallas import tpu_sc as plsc

D_MODEL = 64
NUM_CORES = 2
NUM_SUBCORES = 16
NUM_WORKERS = NUM_CORES * NUM_SUBCORES
GROUP = 128
NBUF = 4
LANES = 16


@functools.lru_cache(maxsize=None)
def _build(n_t: int, n_a: int):
    a_tiles = n_a // GROUP
    f_tiles = D_MODEL // 8

    mesh = plsc.VectorSubcoreMesh(
        core_axis_name="c",
        subcore_axis_name="s",
        num_cores=NUM_CORES,
        num_subcores=NUM_SUBCORES,
    )

    @functools.partial(
        pl.kernel,
        out_type=jax.ShapeDtypeStruct(
            (n_t, f_tiles, a_tiles, 8, GROUP), jnp.float32
        ),
        mesh=mesh,
        scratch_types=[
            pltpu.VMEM((n_t, GROUP), jnp.int32),
            pltpu.VMEM((NBUF, GROUP, D_MODEL), jnp.float32),
            pltpu.VMEM((NBUF, f_tiles, 8, GROUP), jnp.float32),
            pltpu.SemaphoreType.DMA((NBUF,)),
            pltpu.SemaphoreType.DMA((NBUF,)),
        ],
        compiler_params=pltpu.CompilerParams(
            use_tc_tiling_on_sc=False, needs_layout_passes=False
        ),
    )
    def emb_kernel(xt_hbm, table_hbm, out_hbm, idx_v, rows_v, tbuf_v, sem_g, sem_w):
        wid = lax.axis_index("s") * NUM_CORES + lax.axis_index("c")

        pltpu.sync_copy(
            xt_hbm.at[pl.ds(0, n_t), pl.ds(wid * GROUP, GROUP)], idx_v
        )

        def fire_gather(t, buf):
            pltpu.async_copy(
                table_hbm.at[idx_v.at[t]], rows_v.at[buf], sem_g.at[buf]
            )

        def wait_gather(buf):
            pltpu.make_async_copy(
                table_hbm.at[idx_v.at[0]], rows_v.at[buf], sem_g.at[buf]
            ).wait()

        def fire_write(t, buf):
            pltpu.async_copy(
                tbuf_v.at[buf], out_hbm.at[t, pl.ds(0, f_tiles), wid],
                sem_w.at[buf],
            )

        def wait_write(buf):
            pltpu.make_async_copy(
                tbuf_v.at[buf], out_hbm.at[0, pl.ds(0, f_tiles), 0],
                sem_w.at[buf],
            ).wait()

        for b in range(NBUF):
            fire_gather(b, b)

        base = lax.iota(jnp.int32, LANES)
        fdiag = [(base + d) & (LANES - 1) for d in range(LANES)]
        fdiag_lo = [fd & 7 for fd in fdiag]

        @pl.loop(0, n_t // NBUF)
        def _outer(ti):
            for b in range(NBUF):
                t = ti * NBUF + b
                wait_gather(b)

                @pl.when(t >= NBUF)
                def _():
                    wait_write(b)

                @pl.loop(0, GROUP // LANES)
                def _cblk(cb):
                    cvec = base + cb * LANES
                    for f0 in range(0, D_MODEL, LANES):
                        for d in range(LANES):
                            fvec = fdiag[d] + f0
                            v = plsc.load_gather(rows_v.at[b], [cvec, fvec])
                            plsc.store_scatter(
                                tbuf_v.at[b],
                                [fvec >> 3, fdiag_lo[d], cvec],
                                v * 8.0,
                            )

                fire_write(t, b)

                @pl.when(t + NBUF < n_t)
                def _():
                    fire_gather(t + NBUF, b)

        for b in range(NBUF):
            wait_write(b)

    return emb_kernel


def kernel(x, table):
    n_a, n_t = x.shape
    xt = jnp.transpose(x).astype(jnp.int32)
    out5 = _build(n_t, n_a)(xt, table)
    return out5.transpose(2, 4, 0, 1, 3).reshape(n_a, n_t, D_MODEL)


# speedup vs baseline: 1.3750x; 1.3750x over previous
import functools

import jax
import jax.numpy as jnp
from jax import lax
from jax.experimental import pallas as pl
from jax.experimental.pallas import tpu as pltpu
from jax.experimental.pallas import tpu_sc as plsc

D_MODEL = 64
NUM_CORES = 2
NUM_SUBCORES = 16
NUM_WORKERS = NUM_CORES * NUM_SUBCORES
GROUP = 128
NBUF = 4
LANES = 16


@functools.lru_cache(maxsize=None)
def _build(n_t: int, n_a: int):
    a_tiles = n_a // GROUP
    f_tiles = D_MODEL // 8

    mesh = plsc.VectorSubcoreMesh(
        core_axis_name="c",
        subcore_axis_name="s",
        num_cores=NUM_CORES,
        num_subcores=NUM_SUBCORES,
    )

    @functools.partial(
        pl.kernel,
        out_type=jax.ShapeDtypeStruct(
            (n_t, f_tiles, a_tiles, 8, GROUP), jnp.float32
        ),
        mesh=mesh,
        scratch_types=[
            pltpu.VMEM((n_t, GROUP), jnp.int32),
            pltpu.VMEM((NBUF, GROUP, D_MODEL), jnp.float32),
            pltpu.VMEM((NBUF, f_tiles, 8, GROUP), jnp.float32),
            pltpu.SemaphoreType.DMA((NBUF,)),
            pltpu.SemaphoreType.DMA((NBUF,)),
        ],
        compiler_params=pltpu.CompilerParams(
            use_tc_tiling_on_sc=False, needs_layout_passes=False
        ),
    )
    def emb_kernel(xt_hbm, table_hbm, out_hbm, idx_v, rows_v, tbuf_v, sem_g, sem_w):
        wid = lax.axis_index("s") * NUM_CORES + lax.axis_index("c")

        pltpu.sync_copy(
            xt_hbm.at[pl.ds(0, n_t), pl.ds(wid * GROUP, GROUP)], idx_v
        )

        def fire_gather(t, buf):
            pltpu.async_copy(
                table_hbm.at[idx_v.at[t]], rows_v.at[buf], sem_g.at[buf]
            )

        def wait_gather(buf):
            pltpu.make_async_copy(
                table_hbm.at[idx_v.at[0]], rows_v.at[buf], sem_g.at[buf]
            ).wait()

        def fire_write(t, buf):
            pltpu.async_copy(
                tbuf_v.at[buf], out_hbm.at[t, pl.ds(0, f_tiles), wid],
                sem_w.at[buf],
            )

        def wait_write(buf):
            pltpu.make_async_copy(
                tbuf_v.at[buf], out_hbm.at[0, pl.ds(0, f_tiles), 0],
                sem_w.at[buf],
            ).wait()

        for b in range(NBUF):
            fire_gather(b, b)

        base = lax.iota(jnp.int32, LANES)
        fdiag = [(base + d) & (LANES - 1) for d in range(LANES)]
        fdiag_lo = [fd & 7 for fd in fdiag]

        @pl.loop(0, n_t // NBUF)
        def _outer(ti):
            for b in range(NBUF):
                t = ti * NBUF + b
                wait_gather(b)

                @pl.when(t >= NBUF)
                def _():
                    wait_write(b)

                @plsc.parallel_loop(0, GROUP // LANES, unroll=4)
                def _cblk(cb):
                    cvec = base + cb * LANES
                    for f0 in range(0, D_MODEL, LANES):
                        for d in range(LANES):
                            fvec = fdiag[d] + f0
                            v = plsc.load_gather(rows_v.at[b], [cvec, fvec])
                            plsc.store_scatter(
                                tbuf_v.at[b],
                                [fvec >> 3, fdiag_lo[d], cvec],
                                v * 8.0,
                            )

                fire_write(t, b)

                @pl.when(t + NBUF < n_t)
                def _():
                    fire_gather(t + NBUF, b)

        for b in range(NBUF):
            wait_write(b)

    return emb_kernel


def kernel(x, table):
    n_a, n_t = x.shape
    xt = jnp.transpose(x).astype(jnp.int32)
    out5 = _build(n_t, n_a)(xt, table)
    return out5.transpose(2, 4, 0, 1, 3).reshape(n_a, n_t, D_MODEL)


# speedup vs baseline: 1.4906x; 1.0841x over previous
import functools

import jax
import jax.numpy as jnp
from jax import lax
from jax.experimental import pallas as pl
from jax.experimental.pallas import tpu as pltpu
from jax.experimental.pallas import tpu_sc as plsc

D_MODEL = 64
NUM_CORES = 2
NUM_SUBCORES = 16
NUM_WORKERS = NUM_CORES * NUM_SUBCORES
GROUP = 128
NBUF = 4
NWBUF = 2
LANES = 16


@functools.lru_cache(maxsize=None)
def _build(n_t: int, n_a: int):
    a_tiles = n_a // GROUP
    f_tiles = D_MODEL // 8

    mesh = plsc.VectorSubcoreMesh(
        core_axis_name="c",
        subcore_axis_name="s",
        num_cores=NUM_CORES,
        num_subcores=NUM_SUBCORES,
    )

    @functools.partial(
        pl.kernel,
        out_type=jax.ShapeDtypeStruct(
            (n_t, f_tiles, a_tiles, 8, GROUP), jnp.float32
        ),
        mesh=mesh,
        scratch_types=[
            pltpu.VMEM((n_t, GROUP), jnp.int32),
            pltpu.VMEM((NBUF, GROUP, GROUP), jnp.float32),
            pltpu.VMEM((NWBUF, f_tiles, 8, GROUP), jnp.float32),
            pltpu.SemaphoreType.DMA((NBUF,)),
            pltpu.SemaphoreType.DMA((NWBUF,)),
        ],
        compiler_params=pltpu.CompilerParams(
            use_tc_tiling_on_sc=False, needs_layout_passes=False
        ),
    )
    def emb_kernel(xt_hbm, table_hbm, out_hbm, idx_v, rows_v, tbuf_v, sem_g, sem_w):
        wid = lax.axis_index("s") * NUM_CORES + lax.axis_index("c")

        pltpu.sync_copy(
            xt_hbm.at[pl.ds(0, n_t), pl.ds(wid * GROUP, GROUP)], idx_v
        )

        def fire_gather(t, buf):
            pltpu.async_copy(
                table_hbm.at[idx_v.at[t]], rows_v.at[buf], sem_g.at[buf]
            )

        def wait_gather(buf):
            pltpu.make_async_copy(
                table_hbm.at[idx_v.at[0]], rows_v.at[buf], sem_g.at[buf]
            ).wait()

        def fire_write(t, buf):
            pltpu.async_copy(
                tbuf_v.at[buf], out_hbm.at[t, pl.ds(0, f_tiles), wid],
                sem_w.at[buf],
            )

        def wait_write(buf):
            pltpu.make_async_copy(
                tbuf_v.at[buf], out_hbm.at[0, pl.ds(0, f_tiles), 0],
                sem_w.at[buf],
            ).wait()

        for b in range(NBUF):
            fire_gather(b, b)

        base = lax.iota(jnp.int32, LANES)
        fdiag = [(base + d) & (LANES - 1) for d in range(LANES)]
        fdiag_lo = [fd & 7 for fd in fdiag]

        @pl.loop(0, n_t // NBUF)
        def _outer(ti):
            for b in range(NBUF):
                t = ti * NBUF + b
                bw = b % NWBUF
                wait_gather(b)

                @pl.when(t >= NWBUF)
                def _():
                    wait_write(bw)

                @plsc.parallel_loop(0, GROUP // LANES, unroll=4)
                def _cblk(cb):
                    cvec = base + cb * LANES
                    for f0 in range(0, D_MODEL, LANES):
                        for d in range(LANES):
                            fvec = fdiag[d] + f0
                            v = plsc.load_gather(rows_v.at[b], [cvec, fvec])
                            plsc.store_scatter(
                                tbuf_v.at[bw],
                                [fvec >> 3, fdiag_lo[d], cvec],
                                v * 8.0,
                            )

                fire_write(t, bw)

                @pl.when(t + NBUF < n_t)
                def _():
                    fire_gather(t + NBUF, b)

        for b in range(NWBUF):
            wait_write(b)

    return emb_kernel


def kernel(x, table):
    n_a, n_t = x.shape
    xt = jnp.transpose(x).astype(jnp.int32)
    tpad = jnp.pad(table, ((0, 0), (0, 128 - D_MODEL)))
    out5 = _build(n_t, n_a)(xt, tpad)
    return out5.transpose(2, 4, 0, 1, 3).reshape(n_a, n_t, D_MODEL)


# speedup vs baseline: 2.1164x; 1.4198x over previous
import functools

import jax
import jax.numpy as jnp
from jax import lax
from jax.experimental import pallas as pl
from jax.experimental.pallas import tpu as pltpu
from jax.experimental.pallas import tpu_sc as plsc

D_MODEL = 64
NUM_CORES = 2
NUM_SUBCORES = 16
NUM_WORKERS = NUM_CORES * NUM_SUBCORES
GROUP = 128
NBUF = 4
NWBUF = 2
LANES = 16


@functools.lru_cache(maxsize=None)
def _build_transpose(vocab: int):
    nfull = vocab // GROUP
    rem = vocab - nfull * GROUP
    per_w = nfull // NUM_WORKERS
    extra = nfull - per_w * NUM_WORKERS

    mesh = plsc.VectorSubcoreMesh(
        core_axis_name="c",
        subcore_axis_name="s",
        num_cores=NUM_CORES,
        num_subcores=NUM_SUBCORES,
    )

    NSRC = 4
    NDST = 2

    @functools.partial(
        pl.kernel,
        out_type=jax.ShapeDtypeStruct((vocab, GROUP), jnp.float32),
        mesh=mesh,
        scratch_types=[
            pltpu.VMEM((NSRC, D_MODEL, GROUP), jnp.float32),
            pltpu.VMEM((NDST, GROUP, GROUP), jnp.float32),
            pltpu.SemaphoreType.DMA((NSRC,)),
            pltpu.SemaphoreType.DMA((NDST,)),
        ],
        compiler_params=pltpu.CompilerParams(
            use_tc_tiling_on_sc=True, needs_layout_passes=False
        ),
    )
    def tr_kernel(tt_hbm, rem_hbm, tp_hbm, sbuf_v, dbuf_v, sem_r, sem_w):
        wid = lax.axis_index("s") * NUM_CORES + lax.axis_index("c")
        nj = per_w + jnp.where(wid < extra, 1, 0)
        j0 = wid * per_w + jnp.minimum(wid, extra)

        def fire_read(j, buf):
            pltpu.async_copy(
                tt_hbm.at[pl.ds(0, D_MODEL), pl.ds((j0 + j) * GROUP, GROUP)],
                sbuf_v.at[buf], sem_r.at[buf],
            )

        def wait_read(buf):
            pltpu.make_async_copy(
                tt_hbm.at[pl.ds(0, D_MODEL), pl.ds(0, GROUP)],
                sbuf_v.at[buf], sem_r.at[buf],
            ).wait()

        def fire_write(j, buf):
            pltpu.async_copy(
                dbuf_v.at[buf],
                tp_hbm.at[pl.ds((j0 + j) * GROUP, GROUP)],
                sem_w.at[buf],
            )

        def wait_write(buf):
            pltpu.make_async_copy(
                dbuf_v.at[buf], tp_hbm.at[pl.ds(0, GROUP)], sem_w.at[buf]
            ).wait()

        for b in range(NSRC):
            @pl.when(b < nj)
            def _():
                fire_read(b, b)

        base = lax.iota(jnp.int32, LANES)
        fdiag = [(base + d) & (LANES - 1) for d in range(LANES)]

        @pl.loop(0, per_w + 1)
        def _blk(j):
            @pl.when(j < nj)
            def _():
                bs = j % NSRC
                bd = j % NDST
                wait_read(bs)

                @pl.when(j >= NDST)
                def _():
                    wait_write(bd)

                @plsc.parallel_loop(0, GROUP // LANES, unroll=4)
                def _cblk(cb):
                    cvec = base + cb * LANES
                    for f0 in range(0, D_MODEL, LANES):
                        for d in range(LANES):
                            fvec = fdiag[d] + f0
                            v = plsc.load_gather(sbuf_v.at[bs], [fvec, cvec])
                            plsc.store_scatter(dbuf_v.at[bd], [cvec, fvec], v)

                fire_write(j, bd)

                @pl.when(j + NSRC < nj)
                def _():
                    fire_read(j + NSRC, bs)

        for b in range(NDST):
            wait_write(b)

        if rem:
            @pl.when(wid == NUM_WORKERS - 1)
            def _():
                pltpu.sync_copy(
                    rem_hbm.at[pl.ds(0, D_MODEL)],
                    sbuf_v.at[0],
                )
                pltpu.sync_copy(
                    sbuf_v.at[0, pl.ds(0, rem)],
                    tp_hbm.at[pl.ds(nfull * GROUP, rem)],
                )

    return tr_kernel


@functools.lru_cache(maxsize=None)
def _build(n_t: int, n_a: int):
    a_tiles = n_a // GROUP
    f_tiles = D_MODEL // 8

    mesh = plsc.VectorSubcoreMesh(
        core_axis_name="c",
        subcore_axis_name="s",
        num_cores=NUM_CORES,
        num_subcores=NUM_SUBCORES,
    )

    @functools.partial(
        pl.kernel,
        out_type=jax.ShapeDtypeStruct(
            (n_t, f_tiles, a_tiles, 8, GROUP), jnp.float32
        ),
        mesh=mesh,
        scratch_types=[
            pltpu.VMEM((n_t, GROUP), jnp.int32),
            pltpu.VMEM((NBUF, GROUP, GROUP), jnp.float32),
            pltpu.VMEM((NWBUF, f_tiles, 8, GROUP), jnp.float32),
            pltpu.SemaphoreType.DMA((NBUF,)),
            pltpu.SemaphoreType.DMA((NWBUF,)),
        ],
        compiler_params=pltpu.CompilerParams(
            use_tc_tiling_on_sc=True, needs_layout_passes=False
        ),
    )
    def emb_kernel(xt_hbm, table_hbm, out_hbm, idx_v, rows_v, tbuf_v, sem_g, sem_w):
        wid = lax.axis_index("s") * NUM_CORES + lax.axis_index("c")

        pltpu.sync_copy(
            xt_hbm.at[pl.ds(0, n_t), pl.ds(wid * GROUP, GROUP)], idx_v
        )

        def fire_gather(t, buf):
            pltpu.async_copy(
                table_hbm.at[idx_v.at[t]], rows_v.at[buf], sem_g.at[buf]
            )

        def wait_gather(buf):
            pltpu.make_async_copy(
                table_hbm.at[idx_v.at[0]], rows_v.at[buf], sem_g.at[buf]
            ).wait()

        def fire_write(t, buf):
            pltpu.async_copy(
                tbuf_v.at[buf], out_hbm.at[t, pl.ds(0, f_tiles), wid],
                sem_w.at[buf],
            )

        def wait_write(buf):
            pltpu.make_async_copy(
                tbuf_v.at[buf], out_hbm.at[0, pl.ds(0, f_tiles), 0],
                sem_w.at[buf],
            ).wait()

        for b in range(NBUF):
            fire_gather(b, b)

        base = lax.iota(jnp.int32, LANES)
        fdiag = [(base + d) & (LANES - 1) for d in range(LANES)]
        fdiag_lo = [fd & 7 for fd in fdiag]

        @pl.loop(0, n_t // NBUF)
        def _outer(ti):
            for b in range(NBUF):
                t = ti * NBUF + b
                bw = b % NWBUF
                wait_gather(b)

                @pl.when(t >= NWBUF)
                def _():
                    wait_write(bw)

                @plsc.parallel_loop(0, GROUP // LANES, unroll=4)
                def _cblk(cb):
                    cvec = base + cb * LANES
                    for f0 in range(0, D_MODEL, LANES):
                        for d in range(LANES):
                            fvec = fdiag[d] + f0
                            v = plsc.load_gather(rows_v.at[b], [cvec, fvec])
                            plsc.store_scatter(
                                tbuf_v.at[bw],
                                [fvec >> 3, fdiag_lo[d], cvec],
                                v * 8.0,
                            )

                fire_write(t, bw)

                @pl.when(t + NBUF < n_t)
                def _():
                    fire_gather(t + NBUF, b)

        for b in range(NWBUF):
            wait_write(b)

    return emb_kernel


def kernel(x, table):
    n_a, n_t = x.shape
    xt = jnp.transpose(x).astype(jnp.int32)
    vocab = table.shape[0]
    nfull = vocab // GROUP
    rem = vocab - nfull * GROUP
    rem_in = jnp.pad(
        table[nfull * GROUP:, :],
        ((0, GROUP - rem), (0, GROUP - D_MODEL)),
    )
    tp = _build_transpose(vocab)(jnp.transpose(table), rem_in)
    out5 = _build(n_t, n_a)(xt, tp)
    return out5.transpose(2, 4, 0, 1, 3).reshape(n_a, n_t, D_MODEL)


# speedup vs baseline: 2.4764x; 1.1701x over previous
import functools

import jax
import jax.numpy as jnp
from jax import lax
from jax.experimental import pallas as pl
from jax.experimental.pallas import tpu as pltpu
from jax.experimental.pallas import tpu_sc as plsc

D_MODEL = 64
NUM_CORES = 2
NUM_SUBCORES = 16
NUM_WORKERS = NUM_CORES * NUM_SUBCORES
GROUP = 128
NBUF = 4
NWBUF = 2
LANES = 16


@functools.lru_cache(maxsize=None)
def _build_transpose(vocab: int):
    nfull = vocab // GROUP
    rem = vocab - nfull * GROUP
    per_w = nfull // NUM_WORKERS
    extra = nfull - per_w * NUM_WORKERS

    mesh = plsc.VectorSubcoreMesh(
        core_axis_name="c",
        subcore_axis_name="s",
        num_cores=NUM_CORES,
        num_subcores=NUM_SUBCORES,
    )

    NSRC = 4
    NDST = 2

    @functools.partial(
        pl.kernel,
        out_type=jax.ShapeDtypeStruct((vocab // 2, GROUP), jnp.float32),
        mesh=mesh,
        scratch_types=[
            pltpu.VMEM((NSRC, D_MODEL, GROUP), jnp.float32),
            pltpu.VMEM((NDST, D_MODEL, GROUP), jnp.float32),
            pltpu.SemaphoreType.DMA((NSRC,)),
            pltpu.SemaphoreType.DMA((NDST,)),
        ],
        compiler_params=pltpu.CompilerParams(
            use_tc_tiling_on_sc=True, needs_layout_passes=False
        ),
    )
    def tr_kernel(tt_hbm, rem_hbm, tp_hbm, sbuf_v, dbuf_v, sem_r, sem_w):
        wid = lax.axis_index("s") * NUM_CORES + lax.axis_index("c")
        nj = per_w + jnp.where(wid < extra, 1, 0)
        j0 = wid * per_w + jnp.minimum(wid, extra)

        def fire_read(j, buf):
            pltpu.async_copy(
                tt_hbm.at[pl.ds(0, D_MODEL), pl.ds((j0 + j) * GROUP, GROUP)],
                sbuf_v.at[buf], sem_r.at[buf],
            )

        def wait_read(buf):
            pltpu.make_async_copy(
                tt_hbm.at[pl.ds(0, D_MODEL), pl.ds(0, GROUP)],
                sbuf_v.at[buf], sem_r.at[buf],
            ).wait()

        def fire_write(j, buf):
            pltpu.async_copy(
                dbuf_v.at[buf],
                tp_hbm.at[pl.ds((j0 + j) * (GROUP // 2), GROUP // 2)],
                sem_w.at[buf],
            )

        def wait_write(buf):
            pltpu.make_async_copy(
                dbuf_v.at[buf], tp_hbm.at[pl.ds(0, GROUP // 2)], sem_w.at[buf]
            ).wait()

        for b in range(NSRC):
            @pl.when(b < nj)
            def _():
                fire_read(b, b)

        base = lax.iota(jnp.int32, LANES)
        fdiag = [(base + d) & (LANES - 1) for d in range(LANES)]

        @pl.loop(0, per_w + 1)
        def _blk(j):
            @pl.when(j < nj)
            def _():
                bs = j % NSRC
                bd = j % NDST
                wait_read(bs)

                @pl.when(j >= NDST)
                def _():
                    wait_write(bd)

                @plsc.parallel_loop(0, GROUP // LANES, unroll=4)
                def _cblk(cb):
                    cvec = base + cb * LANES
                    chi = cvec >> 1
                    cpar = (cvec & 1) << 6
                    for f0 in range(0, D_MODEL, LANES):
                        for d in range(LANES):
                            fvec = fdiag[d] + f0
                            v = plsc.load_gather(sbuf_v.at[bs], [fvec, cvec])
                            plsc.store_scatter(
                                dbuf_v.at[bd], [chi, fvec + cpar], v
                            )

                fire_write(j, bd)

                @pl.when(j + NSRC < nj)
                def _():
                    fire_read(j + NSRC, bs)

        for b in range(NDST):
            wait_write(b)

        if rem:
            @pl.when(wid == NUM_WORKERS - 1)
            def _():
                pltpu.sync_copy(
                    rem_hbm, sbuf_v.at[0, pl.ds(0, rem // 2)]
                )
                pltpu.sync_copy(
                    sbuf_v.at[0, pl.ds(0, rem // 2)],
                    tp_hbm.at[pl.ds(nfull * (GROUP // 2), rem // 2)],
                )

    return tr_kernel


@functools.lru_cache(maxsize=None)
def _build(n_t: int, n_a: int):
    a_tiles = n_a // GROUP
    f_tiles = D_MODEL // 8

    mesh = plsc.VectorSubcoreMesh(
        core_axis_name="c",
        subcore_axis_name="s",
        num_cores=NUM_CORES,
        num_subcores=NUM_SUBCORES,
    )

    @functools.partial(
        pl.kernel,
        out_type=jax.ShapeDtypeStruct(
            (n_t, f_tiles, a_tiles, 8, GROUP), jnp.float32
        ),
        mesh=mesh,
        scratch_types=[
            pltpu.VMEM((n_t, GROUP), jnp.int32),
            pltpu.VMEM((NBUF, GROUP), jnp.int32),
            pltpu.VMEM((NBUF, GROUP, GROUP), jnp.float32),
            pltpu.VMEM((NWBUF, f_tiles, 8, GROUP), jnp.float32),
            pltpu.SemaphoreType.DMA((NBUF,)),
            pltpu.SemaphoreType.DMA((NWBUF,)),
        ],
        compiler_params=pltpu.CompilerParams(
            use_tc_tiling_on_sc=True, needs_layout_passes=False
        ),
    )
    def emb_kernel(xt_hbm, table_hbm, out_hbm, idx_v, idx2_v, rows_v, tbuf_v,
                   sem_g, sem_w):
        wid = lax.axis_index("s") * NUM_CORES + lax.axis_index("c")

        pltpu.sync_copy(
            xt_hbm.at[pl.ds(0, n_t), pl.ds(wid * GROUP, GROUP)], idx_v
        )

        def fire_gather(t, buf):
            @plsc.parallel_loop(0, GROUP // LANES, unroll=4)
            def _half(cb):
                sl = pl.ds(cb * LANES, LANES)
                idx2_v[buf, sl] = idx_v[t, sl] >> 1

            pltpu.async_copy(
                table_hbm.at[idx2_v.at[buf]], rows_v.at[buf], sem_g.at[buf]
            )

        def wait_gather(buf):
            pltpu.make_async_copy(
                table_hbm.at[idx2_v.at[0]], rows_v.at[buf], sem_g.at[buf]
            ).wait()

        def fire_write(t, buf):
            pltpu.async_copy(
                tbuf_v.at[buf], out_hbm.at[t, pl.ds(0, f_tiles), wid],
                sem_w.at[buf],
            )

        def wait_write(buf):
            pltpu.make_async_copy(
                tbuf_v.at[buf], out_hbm.at[0, pl.ds(0, f_tiles), 0],
                sem_w.at[buf],
            ).wait()

        for b in range(NBUF):
            fire_gather(b, b)

        base = lax.iota(jnp.int32, LANES)
        fdiag = [(base + d) & (LANES - 1) for d in range(LANES)]
        fdiag_lo = [fd & 7 for fd in fdiag]

        @pl.loop(0, n_t // NBUF)
        def _outer(ti):
            for b in range(NBUF):
                t = ti * NBUF + b
                bw = b % NWBUF
                wait_gather(b)

                @pl.when(t >= NWBUF)
                def _():
                    wait_write(bw)

                @plsc.parallel_loop(0, GROUP // LANES, unroll=4)
                def _cblk(cb):
                    cvec = base + cb * LANES
                    pvec = (idx_v[t, pl.ds(cb * LANES, LANES)] & 1) << 6
                    for f0 in range(0, D_MODEL, LANES):
                        for d in range(LANES):
                            fvec = fdiag[d] + f0
                            v = plsc.load_gather(
                                rows_v.at[b], [cvec, fvec + pvec]
                            )
                            plsc.store_scatter(
                                tbuf_v.at[bw],
                                [fvec >> 3, fdiag_lo[d], cvec],
                                v * 8.0,
                            )

                fire_write(t, bw)

                @pl.when(t + NBUF < n_t)
                def _():
                    fire_gather(t + NBUF, b)

        for b in range(NWBUF):
            wait_write(b)

    return emb_kernel


def kernel(x, table):
    n_a, n_t = x.shape
    xt = jnp.transpose(x).astype(jnp.int32)
    vocab = table.shape[0]
    nfull = vocab // GROUP
    rem = vocab - nfull * GROUP
    rem_in = table[nfull * GROUP:, :].reshape(rem // 2, 2 * D_MODEL)
    tp = _build_transpose(vocab)(jnp.transpose(table), rem_in)
    out5 = _build(n_t, n_a)(xt, tp)
    return out5.transpose(2, 4, 0, 1, 3).reshape(n_a, n_t, D_MODEL)
